# baseline (speedup 1.0000x reference)
import functools

import jax
import jax.numpy as jnp
from jax.experimental import pallas as pl
from jax.experimental.pallas import tpu as pltpu

N = 20000
C = 81
P = 128
SPLITS = 10
BLK = N // SPLITS


def _body(*refs):
    prob_refs = refs[:SPLITS]
    lab_refs = refs[SPLITS:2 * SPLITS]
    w_refs = refs[2 * SPLITS:3 * SPLITS]
    pcl_ref, pcp_ref, imw_ref, iml_ref, out_ref = refs[3 * SPLITS:]

    e0 = (jax.lax.broadcasted_iota(jnp.int32, (1, C), 1) == 0).astype(
        jnp.float32)
    bg_active = (iml_ref[0, 0] != 0.0).astype(jnp.float32)

    bg = jnp.zeros((1, 1), dtype=jnp.float32)
    for k in range(SPLITS):
        col = jax.lax.dot_general(
            e0, prob_refs[k][...], (((1,), (1,)), ((), ())),
            preferred_element_type=jnp.float32)
        mask = (lab_refs[k][0] == 0).astype(jnp.float32)
        bg = bg - jnp.sum(mask * w_refs[k][0] * jnp.log(col),
                          axis=(0, 1), keepdims=True)
    bg = bg * bg_active

    pcl = pcl_ref[...]
    iota_c = jax.lax.broadcasted_iota(jnp.int32, (C, P), 0)
    onehot = (iota_c == pcl).astype(jnp.float32)
    gathered = jax.lax.dot_general(
        iml_ref[...], onehot, (((1,), (0,)), ((), ())),
        preferred_element_type=jnp.float32)
    fg_active = (gathered != 0.0) & (pcl > 0)
    fg_vals = imw_ref[...] * jnp.log(pcp_ref[...])
    fg = -jnp.sum(jnp.where(fg_active, fg_vals, 0.0), axis=(0, 1),
                  keepdims=True)

    out_ref[...] = (bg + fg) * (1.0 / N)


def _block(k):
    return pl.BlockSpec((BLK, C), lambda i, k=k: (k, 0))


def _row(k):
    return pl.BlockSpec((1, 1, BLK), lambda i, k=k: (k, 0, 0))


@functools.partial(jax.jit, static_argnames=())
def kernel(pcl_prob, labels, cls_loss_weights, gt_assignment, pc_labels,
           pc_probs, pc_count, img_cls_loss_weights, im_labels_real):
    del gt_assignment, pc_count
    lab2 = labels.reshape(SPLITS, 1, BLK)
    w2 = cls_loss_weights.reshape(SPLITS, 1, BLK)
    out = pl.pallas_call(
        _body,
        grid=(1,),
        in_specs=(
            [_block(k) for k in range(SPLITS)]
            + [_row(k) for k in range(SPLITS)]
            + [_row(k) for k in range(SPLITS)]
            + [
                pl.BlockSpec((1, P), lambda i: (0, 0)),
                pl.BlockSpec((1, P), lambda i: (0, 0)),
                pl.BlockSpec((1, P), lambda i: (0, 0)),
                pl.BlockSpec((1, C), lambda i: (0, 0)),
            ]
        ),
        out_specs=pl.BlockSpec((1, 1), lambda i: (0, 0)),
        out_shape=jax.ShapeDtypeStruct((1, 1), jnp.float32),
    )(*([pcl_prob] * SPLITS), *([lab2] * SPLITS), *([w2] * SPLITS),
      pc_labels.reshape(1, P), pc_probs.reshape(1, P),
      img_cls_loss_weights.reshape(1, P), im_labels_real.reshape(1, C))
    return out[0, 0]
